# baseline (speedup 1.0000x reference)
import functools

import jax
import jax.numpy as jnp
from jax import lax
from jax.experimental import pallas as pl
from jax.experimental.pallas import tpu as pltpu
from jax.experimental.pallas import tpu_sc as plsc

NC = 2
NS = 16
L = 16
NW = NC * NS
EB = 64
NBUF = 4
LOOK = 2
SB = 32


def _mm_body(x_ref, w_ref, o_ref):
    o_ref[...] = jnp.dot(x_ref[...], w_ref[...],
                         preferred_element_type=jnp.float32)


def _matmul(x, W):
    n, d_in = x.shape
    d_out = W.shape[1]
    bm = 1000 if n % 1000 == 0 else 8
    n_pad = ((n + bm - 1) // bm) * bm
    if n_pad != n:
        x = jnp.pad(x, ((0, n_pad - n), (0, 0)))
    h = pl.pallas_call(
        _mm_body,
        grid=(n_pad // bm,),
        in_specs=[
            pl.BlockSpec((bm, d_in), lambda i: (i, 0)),
            pl.BlockSpec((d_in, d_out), lambda i: (0, 0)),
        ],
        out_specs=pl.BlockSpec((bm, d_out), lambda i: (i, 0)),
        out_shape=jax.ShapeDtypeStruct((n_pad, d_out), jnp.float32),
    )(x, W)
    return h[:n] if n_pad != n else h


def _comb_body(p0_ref, p1_ref, b_ref, o_ref):
    o_ref[...] = p0_ref[0] + p1_ref[0] + b_ref[...]


def _combine(part, b, n):
    _, n_acc, d = part.shape
    bm = 1000 if n % 1000 == 0 else 8
    n_pad = ((n + bm - 1) // bm) * bm
    assert n_pad <= n_acc
    out = pl.pallas_call(
        _comb_body,
        grid=(n_pad // bm,),
        in_specs=[
            pl.BlockSpec((1, bm, d), lambda i: (0, i, 0)),
            pl.BlockSpec((1, bm, d), lambda i: (1, i, 0)),
            pl.BlockSpec((1, d), lambda i: (0, 0)),
        ],
        out_specs=pl.BlockSpec((bm, d), lambda i: (i, 0)),
        out_shape=jax.ShapeDtypeStruct((n_pad, d), jnp.float32),
    )(part, part, b.reshape(1, d))
    return out[:n] if n_pad != n else out


def _make_edge_kernel(n_acc, d, nsb):
    rows_per_tile = n_acc // NS
    mesh = plsc.VectorSubcoreMesh(core_axis_name="c", subcore_axis_name="s")
    scratch = [
        pltpu.VMEM_SHARED((n_acc, d), jnp.float32),
        pltpu.VMEM((SB, EB), jnp.int32),
        pltpu.VMEM((SB, EB), jnp.int32),
        pltpu.VMEM((SB, EB), jnp.float32),
        pltpu.VMEM((NBUF, EB, d), jnp.float32),
    ] + [pltpu.SemaphoreType.DMA] * (2 * NBUF)

    @functools.partial(
        pl.kernel,
        mesh=mesh,
        out_type=jax.ShapeDtypeStruct((NC, n_acc, d), jnp.float32),
        scratch_types=scratch,
    )
    def edge_kernel(h_hbm, src_hbm, dst_hbm, w_hbm, part_hbm,
                    acc, src_a, dst_a, w_a, rows, *sems):
        gsem = sems[:NBUF]
        ssem = sems[NBUF:]
        cid = lax.axis_index("c")
        sid = lax.axis_index("s")
        wid = sid * NC + cid

        @pl.loop(0, EB)
        def _zero_rows(r):
            for j in range(d // L):
                rows[0, r, pl.ds(j * L, L)] = jnp.zeros((L,), jnp.float32)

        stripe0 = sid * rows_per_tile
        done = 0
        while done < rows_per_tile:
            step = min(EB, rows_per_tile - done)
            pltpu.sync_copy(rows.at[0, pl.ds(0, step)],
                            acc.at[pl.ds(stripe0 + done, step)])
            done += step
        plsc.subcore_barrier()

        @pl.loop(0, nsb)
        def _super(sb):
            pltpu.sync_copy(src_hbm.at[wid, sb], src_a)
            pltpu.sync_copy(dst_hbm.at[wid, sb], dst_a)
            pltpu.sync_copy(w_hbm.at[wid, sb], w_a)

            for j in range(LOOK):
                pltpu.async_copy(h_hbm.at[src_a.at[j]], rows.at[j], gsem[j])

            @pl.loop(0, SB, step=NBUF)
            def _batches(i):
                for t in range(NBUF):
                    j = i + t
                    bn = (t + LOOK) % NBUF
                    jn = j + LOOK

                    @pl.when(jn < SB)
                    def _issue_next():
                        jp = jn - NBUF

                        @pl.when(jp >= 0)
                        def _wait_prev_scatter():
                            pltpu.make_async_copy(
                                rows.at[bn], acc.at[dst_a.at[jp]], ssem[bn]
                            ).wait()

                        pltpu.async_copy(h_hbm.at[src_a.at[jn]],
                                         rows.at[bn], gsem[bn])

                    pltpu.make_async_copy(h_hbm.at[src_a.at[j]],
                                          rows.at[t], gsem[t]).wait()

                    @pl.loop(0, EB // L)
                    def _scale(g):
                        wchunk = w_a[j, pl.ds(g * L, L)]
                        for k in range(L):
                            wv = jnp.full((L,), wchunk[k], jnp.float32)
                            e = g * L + k
                            for f in range(d // L):
                                rows[t, e, pl.ds(f * L, L)] = (
                                    rows[t, e, pl.ds(f * L, L)] * wv)

                    pltpu.async_copy(rows.at[t], acc.at[dst_a.at[j]],
                                     ssem[t], add=True)

            for t in range(NBUF):
                pltpu.make_async_copy(
                    rows.at[t], acc.at[dst_a.at[SB - NBUF + t]], ssem[t]
                ).wait()

        plsc.subcore_barrier()
        done = 0
        while done < rows_per_tile:
            step = min(EB, rows_per_tile - done)
            pltpu.sync_copy(acc.at[pl.ds(stripe0 + done, step)],
                            part_hbm.at[cid, pl.ds(stripe0 + done, step)])
            done += step

    return edge_kernel


def kernel(x, edge_index, edge_weight, W, b):
    n, d_in = x.shape
    d = W.shape[1]
    e = edge_index.shape[1]

    h = _matmul(x, W)

    chunk = NW * EB * SB
    e_pad = ((e + chunk - 1) // chunk) * chunk
    src = edge_index[1]
    dst = edge_index[0]
    w = edge_weight
    if e_pad != e:
        pad = e_pad - e
        src = jnp.pad(src, (0, pad))
        dst = jnp.pad(dst, (0, pad))
        w = jnp.pad(w, (0, pad))
    nsb = e_pad // (NW * EB * SB)
    src = src.reshape(NW, nsb, SB, EB)
    dst = dst.reshape(NW, nsb, SB, EB)
    w = w.reshape(NW, nsb, SB, EB)

    n_acc = ((n + NS * 8 - 1) // (NS * 8)) * (NS * 8)

    part = _make_edge_kernel(n_acc, d, nsb)(h, src, dst, w)
    return _combine(part, b, n)
